# baseline (speedup 1.0000x reference)
import functools

import jax
import jax.numpy as jnp
from jax import lax
from jax.experimental import pallas as pl
from jax.experimental.pallas import tpu as pltpu
from jax.experimental.pallas import tpu_sc as plsc

_NC = 2
_NS = 16
_B0 = 544
_B1 = 480


def _gather_call(labels, table, batch, hidden):
    assert _NS * (_B0 + _B1) == batch

    mesh = plsc.VectorSubcoreMesh(core_axis_name="c", subcore_axis_name="s")

    @functools.partial(
        pl.kernel,
        mesh=mesh,
        out_type=jax.ShapeDtypeStruct((batch, hidden), jnp.float32),
        scratch_types=[
            pltpu.VMEM((_B0,), jnp.int32),
            pltpu.VMEM((_B0, hidden), jnp.float32),
            pltpu.SemaphoreType.DMA,
        ],
    )
    def gather_kernel(labels_hbm, table_hbm, out_hbm, idx_v, rows_v, sem):
        c = lax.axis_index("c")
        s = lax.axis_index("s")

        @pl.when(c == 0)
        def _():
            base = s * _B0
            pltpu.sync_copy(labels_hbm.at[pl.ds(base, _B0)], idx_v)
            pltpu.async_copy(table_hbm.at[idx_v], rows_v, sem).wait()
            pltpu.sync_copy(rows_v, out_hbm.at[pl.ds(base, _B0)])

        @pl.when(c == 1)
        def _():
            base = _NS * _B0 + s * _B1
            idx1 = idx_v.at[pl.ds(0, _B1)]
            rows1 = rows_v.at[pl.ds(0, _B1)]
            pltpu.sync_copy(labels_hbm.at[pl.ds(base, _B1)], idx1)
            pltpu.async_copy(table_hbm.at[idx1], rows1, sem).wait()
            pltpu.sync_copy(rows1, out_hbm.at[pl.ds(base, _B1)])

    return gather_kernel(labels, table)


def kernel(labels, train, table):
    del train
    batch = labels.shape[0]
    hidden = table.shape[1]
    return _gather_call(labels.astype(jnp.int32), table, batch, hidden)


# speedup vs baseline: 1.0127x; 1.0127x over previous
import functools

import jax
import jax.numpy as jnp
from jax import lax
from jax.experimental import pallas as pl
from jax.experimental.pallas import tpu as pltpu
from jax.experimental.pallas import tpu_sc as plsc

_NC = 2
_NS = 16
_B0 = 480
_B1 = 544


def _gather_call(labels, table, batch, hidden):
    assert _NS * (_B0 + _B1) == batch

    mesh = plsc.VectorSubcoreMesh(core_axis_name="c", subcore_axis_name="s")

    @functools.partial(
        pl.kernel,
        mesh=mesh,
        out_type=jax.ShapeDtypeStruct((batch, hidden), jnp.float32),
        scratch_types=[
            pltpu.VMEM((max(_B0, _B1),), jnp.int32),
            pltpu.VMEM((max(_B0, _B1), hidden), jnp.float32),
            pltpu.SemaphoreType.DMA,
        ],
    )
    def gather_kernel(labels_hbm, table_hbm, out_hbm, idx_v, rows_v, sem):
        c = lax.axis_index("c")
        s = lax.axis_index("s")

        @pl.when(c == 0)
        def _():
            base = s * _B0
            idx0 = idx_v.at[pl.ds(0, _B0)]
            rows0 = rows_v.at[pl.ds(0, _B0)]
            pltpu.sync_copy(labels_hbm.at[pl.ds(base, _B0)], idx0)
            pltpu.async_copy(table_hbm.at[idx0], rows0, sem).wait()
            pltpu.sync_copy(rows0, out_hbm.at[pl.ds(base, _B0)])

        @pl.when(c == 1)
        def _():
            base = _NS * _B0 + s * _B1
            idx1 = idx_v.at[pl.ds(0, _B1)]
            rows1 = rows_v.at[pl.ds(0, _B1)]
            pltpu.sync_copy(labels_hbm.at[pl.ds(base, _B1)], idx1)
            pltpu.async_copy(table_hbm.at[idx1], rows1, sem).wait()
            pltpu.sync_copy(rows1, out_hbm.at[pl.ds(base, _B1)])

    return gather_kernel(labels, table)


def kernel(labels, train, table):
    del train
    batch = labels.shape[0]
    hidden = table.shape[1]
    return _gather_call(labels.astype(jnp.int32), table, batch, hidden)


# speedup vs baseline: 1.0178x; 1.0050x over previous
import functools

import jax
import jax.numpy as jnp
from jax import lax
from jax.experimental import pallas as pl
from jax.experimental.pallas import tpu as pltpu
from jax.experimental.pallas import tpu_sc as plsc

_NC = 2
_NS = 16
_NW = _NC * _NS


def _gather_call(labels, table, batch, hidden):
    b_per_w = batch // _NW

    mesh = plsc.VectorSubcoreMesh(core_axis_name="c", subcore_axis_name="s")

    @functools.partial(
        pl.kernel,
        mesh=mesh,
        out_type=jax.ShapeDtypeStruct((batch, hidden), jnp.float32),
        scratch_types=[
            pltpu.VMEM((b_per_w,), jnp.int32),
            pltpu.VMEM((b_per_w, hidden), jnp.float32),
            pltpu.SemaphoreType.DMA,
        ],
    )
    def gather_kernel(labels_hbm, table_hbm, out_hbm, idx_v, rows_v, sem):
        wid = lax.axis_index("s") * _NC + lax.axis_index("c")
        base = wid * b_per_w
        pltpu.sync_copy(labels_hbm.at[pl.ds(base, b_per_w)], idx_v)
        pltpu.async_copy(table_hbm.at[idx_v], rows_v, sem).wait()
        pltpu.sync_copy(rows_v, out_hbm.at[pl.ds(base, b_per_w)])

    return gather_kernel(labels, table)


def kernel(labels, train, table):
    del train
    batch = labels.shape[0]
    hidden = table.shape[1]
    return _gather_call(labels.astype(jnp.int32), table, batch, hidden)
